# baseline (speedup 1.0000x reference)
import functools

import jax
import jax.numpy as jnp
from jax.experimental import pallas as pl
from jax.experimental.pallas import tpu as pltpu
from jax.experimental.pallas import tpu_sc as plsc

B = 1024
D = 64
K = 8192

KB1 = 2048
NK1 = K // KB1
C = 8
HB = 1024
IMAX = 2 ** 31 - 1

GB = C * B
SC_NC = 2
SC_NS = 16
SC_NW = SC_NC * SC_NS
BPW = GB // SC_NW


def _seq_butterfly(sq_rows):
    acc = None
    for j in range(8):
        x = sq_rows[8 * j:8 * j + 8]
        a = (x[0] + x[4]) + (x[2] + x[6])
        b = (x[1] + x[5]) + (x[3] + x[7])
        c = a + b
        acc = c if acc is None else acc + c
    return acc


def _merge2(x1, x2, y1, y2):
    lo = jnp.minimum(x1, y1)
    hi = jnp.minimum(jnp.maximum(x1, y1), jnp.minimum(x2, y2))
    return lo, hi


def _cand_kernel(z_ref, et_ref, cand_ref, m1_scr, m2_scr):
    ki = pl.program_id(0)
    et = et_ref[:]
    etb = et.astype(jnp.bfloat16)
    zb = z_ref[:].astype(jnp.bfloat16)
    g = jax.lax.dot(zb, etb,
                    preferred_element_type=jnp.float32)
    e2 = jnp.sum(et * et, axis=0, keepdims=True)
    s = e2 - (g + g)
    i = jax.lax.bitcast_convert_type(s, jnp.int32)
    key = jnp.where(i >= 0, i, i ^ jnp.int32(0x7FFFFFFF))
    kids = jax.lax.broadcasted_iota(jnp.int32, (1, KB1), 1) + ki * KB1
    key = (key & jnp.int32(-8192)) | kids

    w = KB1
    t1 = key
    t2 = None
    while w > 128:
        h = w // 2
        x, y = t1[:, 0:h], t1[:, h:w]
        if t2 is None:
            t1, t2 = jnp.minimum(x, y), jnp.maximum(x, y)
        else:
            t1, t2 = _merge2(x, t2[:, 0:h], y, t2[:, h:w])
        w = h
    n1, n2 = t1, t2

    @pl.when(ki == 0)
    def _first():
        m1_scr[:] = n1
        m2_scr[:] = n2

    @pl.when(ki > 0)
    def _rest():
        u1, u2 = _merge2(m1_scr[:], m2_scr[:], n1, n2)
        m1_scr[:] = u1
        m2_scr[:] = u2

    @pl.when(ki == NK1 - 1)
    def _extract():
        pool = jnp.concatenate([m1_scr[:], m2_scr[:]], axis=1)
        cands = []
        for _ in range(C):
            m = jnp.min(pool, axis=1, keepdims=True)
            cands.append(m)
            pool = jnp.where(pool == m, IMAX, pool)
        ck = jnp.concatenate(cands, axis=1)
        cand_ref[:] = ck & jnp.int32(0x1FFF)


def _sc_gather(table_hbm, idx_hbm, out_hbm, idx_v, rows_v, sem):
    wid = jax.lax.axis_index("s") * SC_NC + jax.lax.axis_index("c")
    base = wid * BPW
    pltpu.sync_copy(idx_hbm.at[pl.ds(base, BPW)], idx_v)
    pltpu.async_copy(table_hbm.at[idx_v], rows_v, sem).wait()
    pltpu.sync_copy(rows_v, out_hbm.at[pl.ds(base, BPW)])


def _exact_kernel(g_ref, z_ref, cand_ref,
                  idx_ref, zq_ref, commit_ref, perp_ref, d8_scr):
    z = z_ref[:]
    zt = z.T
    cand = cand_ref[:]
    gcs = []
    for c in range(C):
        gf = g_ref[pl.ds(c * B, B), :]
        odd = (cand[:, c:c + 1] & 1) == 1
        gc = jnp.where(odd, gf[:, D:2 * D], gf[:, 0:D])
        gcs.append(gc)
    for c in range(C):
        gc = gcs[c]
        gct = gc.T
        diff = zt - gct
        sq = diff * diff
        rows = [sq[d:d + 1, :] for d in range(D)]
        d8_scr[pl.ds(c, 1), :] = _seq_butterfly(rows)

    dt = d8_scr[:].T
    dmin = jnp.min(dt, axis=1, keepdims=True)
    win = jnp.min(jnp.where(dt == dmin, cand, IMAX), axis=1, keepdims=True)
    idx_ref[:] = win

    q = jnp.zeros((B, D), jnp.float32)
    for c in range(C):
        mask = win == cand[:, c:c + 1]
        q = q + jnp.where(mask, gcs[c], 0.0)
    zq_ref[:] = z + (q - z)
    commit_ref[:] = (0.25 / (B * D)) * jnp.sum(
        (z - q) ** 2, keepdims=True).reshape(1, 1)

    h = jnp.zeros((1, 1), jnp.float32)
    for j in range(K // HB):
        kids = jax.lax.broadcasted_iota(jnp.int32, (1, HB), 1) + j * HB
        oh = (win == kids).astype(jnp.float32)
        counts = jnp.sum(oh, axis=0, keepdims=True)
        p = counts * (1.0 / B)
        h = h + jnp.sum(p * jnp.log(p + 1e-10), keepdims=True).reshape(1, 1)
    perp_ref[:] = jnp.exp(-h)


@functools.partial(jax.jit, static_argnames=())
def kernel(z, embedding):
    emb = embedding.reshape(K, D)
    emb_t = emb.T

    cand = pl.pallas_call(
        _cand_kernel,
        grid=(NK1,),
        in_specs=[
            pl.BlockSpec((B, D), lambda ki: (0, 0)),
            pl.BlockSpec((D, KB1), lambda ki: (0, ki)),
        ],
        out_specs=pl.BlockSpec((B, C), lambda ki: (0, 0)),
        out_shape=jax.ShapeDtypeStruct((B, C), jnp.int32),
        scratch_shapes=[pltpu.VMEM((B, 128), jnp.int32),
                        pltpu.VMEM((B, 128), jnp.int32)],
        compiler_params=pltpu.CompilerParams(
            dimension_semantics=("arbitrary",)),
    )(z, emb_t)

    gidx = cand.T.reshape(GB) >> 1
    emb2 = emb.reshape(K // 2, 2 * D)
    g = pl.kernel(
        _sc_gather,
        out_type=jax.ShapeDtypeStruct((GB, 2 * D), jnp.float32),
        mesh=plsc.VectorSubcoreMesh(core_axis_name="c", subcore_axis_name="s"),
        scratch_types=[
            pltpu.VMEM((BPW,), jnp.int32),
            pltpu.VMEM((BPW, 2 * D), jnp.float32),
            pltpu.SemaphoreType.DMA,
        ],
    )(emb2, gidx)

    idx, zq, commit, perp = pl.pallas_call(
        _exact_kernel,
        in_specs=[
            pl.BlockSpec((GB, 2 * D), lambda: (0, 0)),
            pl.BlockSpec((B, D), lambda: (0, 0)),
            pl.BlockSpec((B, C), lambda: (0, 0)),
        ],
        out_specs=[
            pl.BlockSpec((B, 1), lambda: (0, 0)),
            pl.BlockSpec((B, D), lambda: (0, 0)),
            pl.BlockSpec((1, 1), lambda: (0, 0)),
            pl.BlockSpec((1, 1), lambda: (0, 0)),
        ],
        out_shape=[
            jax.ShapeDtypeStruct((B, 1), jnp.int32),
            jax.ShapeDtypeStruct((B, D), jnp.float32),
            jax.ShapeDtypeStruct((1, 1), jnp.float32),
            jax.ShapeDtypeStruct((1, 1), jnp.float32),
        ],
        scratch_shapes=[
            pltpu.VMEM((C, B), jnp.float32),
        ],
    )(g, z, cand)

    commitment_loss = commit.reshape(())
    codebook_loss = jnp.zeros((), jnp.float32)
    perplexity = perp.reshape(())
    return zq, idx, commitment_loss, codebook_loss, perplexity
